# Initial kernel scaffold; baseline (speedup 1.0000x reference)
import jax
import jax.numpy as jnp
from jax.experimental import pallas as pl


def kernel(x_user, x_item, edge_index_ui, edge_index_iu, W_user, b_user, W_item, b_item, Wl_ui_0, Wr_ui_0, b_ui_0, Wl_iu_0, Wr_iu_0, b_iu_0, Wl_ui_1, Wr_ui_1, b_ui_1, Wl_iu_1, Wr_iu_1, b_iu_1):
    raise NotImplementedError("write your pallas kernel here")


# speedup vs baseline: 2.3483x; 2.3483x over previous
import jax
import jax.numpy as jnp
from jax import lax
from jax.experimental import pallas as pl
from jax.experimental.pallas import tpu as pltpu
from jax.experimental.pallas import tpu_sc as plsc

N = 50000
E = 800000
DF = 128
H = 64
HH = H // 2
NSUB = 16
NPAD = 50048
RPT = NPAD // NSUB
EPT = E // NSUB
CH = 80
NCH = EPT // CH
BN = 2000
GRID = N // BN

_SC_MESH = plsc.VectorSubcoreMesh(core_axis_name="c", subcore_axis_name="s")
_SC_PARAMS = pltpu.CompilerParams(use_tc_tiling_on_sc=False)


def _segsum_body(y_st, src, dst, zrows, seg_st, sidx_v, didx_v, rows_v, sem, acc):
    c = lax.axis_index("c")
    s = lax.axis_index("s")
    r0 = pl.multiple_of(s * RPT, 8)
    half = c * N
    pltpu.sync_copy(zrows, acc.at[pl.ds(r0, RPT)])
    plsc.subcore_barrier()

    def chunk(i, carry):
        base = pl.multiple_of(s * EPT + i * CH, 8)
        pltpu.sync_copy(src.at[pl.ds(base, CH)], sidx_v)
        pltpu.sync_copy(dst.at[pl.ds(base, CH)], didx_v)
        for k in range(CH // 16):
            sl = pl.ds(k * 16, 16)
            sidx_v[sl] = sidx_v[sl] + half
        pltpu.async_copy(y_st.at[sidx_v], rows_v, sem).wait()
        pltpu.sync_copy(rows_v, acc.at[didx_v], add=True)
        return carry

    lax.fori_loop(0, NCH, chunk, 0)
    plsc.subcore_barrier()
    o0 = pl.multiple_of(c * NPAD + r0, 8)
    pltpu.sync_copy(acc.at[pl.ds(r0, RPT)], seg_st.at[pl.ds(o0, RPT)])


_segsum = pl.kernel(
    _segsum_body,
    mesh=_SC_MESH,
    out_type=jax.ShapeDtypeStruct((2 * NPAD, HH), jnp.float32),
    scratch_types=[pltpu.VMEM((CH,), jnp.int32),
                   pltpu.VMEM((CH,), jnp.int32),
                   pltpu.VMEM((CH, HH), jnp.float32),
                   pltpu.SemaphoreType.DMA,
                   pltpu.VMEM_SHARED((NPAD, HH), jnp.float32)],
    compiler_params=_SC_PARAMS,
)


def _counts_body(dst_st, ones_hbm, zrows, cnt_st, didx_v, ones_v, acc):
    c = lax.axis_index("c")
    s = lax.axis_index("s")
    r0 = pl.multiple_of(s * RPT, 8)
    pltpu.sync_copy(zrows, acc.at[pl.ds(r0, RPT)])
    pltpu.sync_copy(ones_hbm, ones_v)
    plsc.subcore_barrier()

    def chunk(i, carry):
        base = pl.multiple_of(c * E + s * EPT + i * CH, 8)
        pltpu.sync_copy(dst_st.at[pl.ds(base, CH)], didx_v)
        pltpu.sync_copy(ones_v, acc.at[didx_v], add=True)
        return carry

    lax.fori_loop(0, NCH, chunk, 0)
    plsc.subcore_barrier()
    o0 = pl.multiple_of(c * NPAD + r0, 8)
    pltpu.sync_copy(acc.at[pl.ds(r0, RPT)], cnt_st.at[pl.ds(o0, RPT)])


_counts = pl.kernel(
    _counts_body,
    mesh=_SC_MESH,
    out_type=jax.ShapeDtypeStruct((2 * NPAD, 16), jnp.float32),
    scratch_types=[pltpu.VMEM((CH,), jnp.int32),
                   pltpu.VMEM((CH, 16), jnp.float32),
                   pltpu.VMEM_SHARED((NPAD, 16), jnp.float32)],
    compiler_params=_SC_PARAMS,
)


def _mm(a, b):
    return jnp.dot(a, b, preferred_element_type=jnp.float32)


def _proj_writes(x_src, x_dst, Wl, Wr, b, y_st, z):
    y = _mm(x_src, Wl)
    y_st[0] = y[:, :HH]
    y_st[1] = y[:, HH:]
    z[...] = _mm(x_dst, Wr) + b


def _enc_proj_body(xu_in, xi_in, Wu, bu, Wi, bi,
                   Wl_ui, Wr_ui, b_ui, Wl_iu, Wr_iu, b_iu,
                   yui_st, yiu_st, zi, zu):
    xu = jax.nn.relu(_mm(xu_in[...], Wu[...]) + bu[...])
    xi = jax.nn.relu(_mm(xi_in[...], Wi[...]) + bi[...])
    _proj_writes(xu, xi, Wl_ui[...], Wr_ui[...], b_ui[...], yui_st, zi)
    _proj_writes(xi, xu, Wl_iu[...], Wr_iu[...], b_iu[...], yiu_st, zu)


def _combine(s_lo, s_hi, cnt, z):
    seg = jnp.concatenate([s_lo[0], s_hi[0]], axis=1)
    return jax.nn.relu(seg / jnp.maximum(cnt[0][:, :1], 1.0) + z[...])


def _comb_proj_body(sui_lo, sui_hi, siu_lo, siu_hi, cnt_ui, cnt_iu, zi_in, zu_in,
                    Wl_ui, Wr_ui, b_ui, Wl_iu, Wr_iu, b_iu,
                    yui_st, yiu_st, zi, zu):
    xi = _combine(sui_lo, sui_hi, cnt_ui, zi_in)
    xu = _combine(siu_lo, siu_hi, cnt_iu, zu_in)
    _proj_writes(xu, xi, Wl_ui[...], Wr_ui[...], b_ui[...], yui_st, zi)
    _proj_writes(xi, xu, Wl_iu[...], Wr_iu[...], b_iu[...], yiu_st, zu)


def _comb_final_body(sui_lo, sui_hi, siu_lo, siu_hi, cnt_ui, cnt_iu, zi_in, zu_in,
                     xu_out, xi_out):
    xi_out[...] = _combine(sui_lo, sui_hi, cnt_ui, zi_in)
    xu_out[...] = _combine(siu_lo, siu_hi, cnt_iu, zu_in)


def _row_spec(w):
    return pl.BlockSpec((BN, w), lambda i: (i, 0))


def _plane_spec(p, w):
    return pl.BlockSpec((1, BN, w), lambda i, _p=p: (_p, i, 0))


def _full_spec(shape):
    return pl.BlockSpec(shape, lambda i: tuple(0 for _ in shape))


_Y_OUT = [jax.ShapeDtypeStruct((2, N, HH), jnp.float32)] * 2
_Z_OUT = [jax.ShapeDtypeStruct((N, H), jnp.float32)] * 2
_W_SPECS = [_full_spec((H, H)), _full_spec((H, H)), _full_spec((1, H))] * 2
_Y_OUT_SPEC = pl.BlockSpec((2, BN, HH), lambda i: (0, i, 0))
_PROJ_OUT_SPECS = [_Y_OUT_SPEC, _Y_OUT_SPEC, _row_spec(H), _row_spec(H)]

_enc_proj = pl.pallas_call(
    _enc_proj_body,
    grid=(GRID,),
    in_specs=[_row_spec(DF), _row_spec(DF),
              _full_spec((DF, H)), _full_spec((1, H)),
              _full_spec((DF, H)), _full_spec((1, H))] + _W_SPECS,
    out_specs=_PROJ_OUT_SPECS,
    out_shape=_Y_OUT + _Z_OUT,
)

_COMB_IN_SPECS = [_plane_spec(0, HH), _plane_spec(1, HH),
                  _plane_spec(0, HH), _plane_spec(1, HH),
                  _plane_spec(0, 16), _plane_spec(1, 16),
                  _row_spec(H), _row_spec(H)]

_comb_proj = pl.pallas_call(
    _comb_proj_body,
    grid=(GRID,),
    in_specs=_COMB_IN_SPECS + _W_SPECS,
    out_specs=_PROJ_OUT_SPECS,
    out_shape=_Y_OUT + _Z_OUT,
)

_comb_final = pl.pallas_call(
    _comb_final_body,
    grid=(GRID,),
    in_specs=_COMB_IN_SPECS,
    out_specs=[_row_spec(H)] * 2,
    out_shape=[jax.ShapeDtypeStruct((N, H), jnp.float32)] * 2,
)


def _seg_pair(y, src, dst, zrows):
    seg = _segsum(y.reshape(2 * N, HH), src, dst, zrows).reshape(2, NPAD, HH)
    return seg, seg


def kernel(x_user, x_item, edge_index_ui, edge_index_iu,
           W_user, b_user, W_item, b_item,
           Wl_ui_0, Wr_ui_0, b_ui_0, Wl_iu_0, Wr_iu_0, b_iu_0,
           Wl_ui_1, Wr_ui_1, b_ui_1, Wl_iu_1, Wr_iu_1, b_iu_1):
    bu = b_user.reshape(1, H)
    bi = b_item.reshape(1, H)
    bui0 = b_ui_0.reshape(1, H)
    biu0 = b_iu_0.reshape(1, H)
    bui1 = b_ui_1.reshape(1, H)
    biu1 = b_iu_1.reshape(1, H)
    zrows = jnp.zeros((RPT, HH), jnp.float32)
    zrows16 = jnp.zeros((RPT, 16), jnp.float32)
    ones16 = jnp.ones((CH, 16), jnp.float32)
    src_ui, dst_ui = edge_index_ui[0], edge_index_ui[1]
    src_iu, dst_iu = edge_index_iu[0], edge_index_iu[1]
    dst_st = jnp.concatenate([dst_ui, dst_iu])

    cnt = _counts(dst_st, ones16, zrows16).reshape(2, NPAD, 16)

    yui, yiu, zi, zu = _enc_proj(
        x_user, x_item, W_user, bu, W_item, bi,
        Wl_ui_0, Wr_ui_0, bui0, Wl_iu_0, Wr_iu_0, biu0)

    sui, _ = _seg_pair(yui, src_ui, dst_ui, zrows)
    siu, _ = _seg_pair(yiu, src_iu, dst_iu, zrows)

    yui, yiu, zi, zu = _comb_proj(
        sui, sui, siu, siu, cnt, cnt, zi, zu,
        Wl_ui_1, Wr_ui_1, bui1, Wl_iu_1, Wr_iu_1, biu1)

    sui, _ = _seg_pair(yui, src_ui, dst_ui, zrows)
    siu, _ = _seg_pair(yiu, src_iu, dst_iu, zrows)

    xu_out, xi_out = _comb_final(
        sui, sui, siu, siu, cnt, cnt, zi, zu)
    return xu_out, xi_out
